# baseline (speedup 1.0000x reference)
import jax
import jax.numpy as jnp
from jax.experimental import pallas as pl

B, H, W, C = 32, 32, 32, 768
E = 64
TOP_K = 8
PH = PW = 2
NPH, NPW = H // PH, W // PW
P = NPH * NPW
NSTREAM = 8
IMGS = NSTREAM // 2
HH = H // 2
R = P * IMGS


def _router_kernel(*refs):
    x_refs = refs[:NSTREAM]
    gw_ref, w_ref, i_ref, l_ref = refs[NSTREAM:]

    gw = gw_ref[...].astype(jnp.bfloat16)
    means_parts = []
    for q in range(NSTREAM):
        xb = x_refs[q][0]
        s = xb[:, :, :C] + xb[:, :, C:]
        s = s.reshape(HH // 2, 2, NPW, C)
        s = s[:, 0, :, :] + s[:, 1, :, :]
        means_parts.append(s.reshape(P // 2, C))
    means = jnp.concatenate(means_parts, axis=0) * 0.25

    logits = jax.lax.dot_general(
        means.astype(jnp.bfloat16), gw,
        dimension_numbers=(((1,), (1,)), ((), ())),
        preferred_element_type=jnp.float32,
    )
    l_ref[...] = logits

    iota_f = jax.lax.broadcasted_iota(jnp.int32, (R, E), 1).astype(jnp.float32)
    vals = logits
    ws, ids = [], []
    neg = jnp.float32(-jnp.inf)
    for _ in range(TOP_K):
        mk = jnp.max(vals, axis=-1, keepdims=True)
        idx = jnp.min(jnp.where(vals >= mk, iota_f, jnp.float32(E)),
                      axis=-1, keepdims=True)
        ws.append(mk)
        ids.append(idx)
        vals = jnp.where(iota_f == idx, neg, vals)
    lcat = jnp.concatenate(ws, axis=-1)
    icat = jnp.concatenate(ids, axis=-1)
    ex = jnp.exp(lcat - lcat[:, :1])
    w_ref[...] = ex / jnp.sum(ex, axis=-1, keepdims=True)
    i_ref[...] = icat.astype(jnp.int32)


@jax.jit
def kernel(x, spatial_shape, gate_w):
    del spatial_shape
    b = x.shape[0]
    grid = (b // IMGS,)
    x5 = x.reshape(b * 2, HH, W // 2, 2 * C)

    def make_spec(q):
        return pl.BlockSpec((1, HH, W // 2, 2 * C),
                            lambda i, q=q: (NSTREAM * i + q, 0, 0, 0))

    out = pl.pallas_call(
        _router_kernel,
        grid=grid,
        in_specs=[make_spec(q) for q in range(NSTREAM)]
        + [pl.BlockSpec((E, C), lambda i: (0, 0))],
        out_specs=[
            pl.BlockSpec((R, TOP_K), lambda i: (i, 0)),
            pl.BlockSpec((R, TOP_K), lambda i: (i, 0)),
            pl.BlockSpec((R, E), lambda i: (i, 0)),
        ],
        out_shape=[
            jax.ShapeDtypeStruct((b * P, TOP_K), jnp.float32),
            jax.ShapeDtypeStruct((b * P, TOP_K), jnp.int32),
            jax.ShapeDtypeStruct((b * P, E), jnp.float32),
        ],
    )(*([x5] * NSTREAM), gate_w)
    return out[0], out[1], out[2]
